# Initial kernel scaffold; baseline (speedup 1.0000x reference)
import jax
import jax.numpy as jnp
from jax.experimental import pallas as pl


def kernel(x, table):
    raise NotImplementedError("write your pallas kernel here")


# speedup vs baseline: 4.9496x; 4.9496x over previous
import functools

import jax
import jax.numpy as jnp
from jax import lax
from jax.experimental import pallas as pl
from jax.experimental.pallas import tpu as pltpu
from jax.experimental.pallas import tpu_sc as plsc

NC = 2
NS = 16
NW = NC * NS

IDX_W = 128
K = 16
CHUNK = K * IDX_W


def _lookup_kernel(B, V, D):
    b_per_w = B // NW
    n_chunks = b_per_w // CHUNK
    mesh = plsc.VectorSubcoreMesh(core_axis_name="c", subcore_axis_name="s")

    @functools.partial(
        pl.kernel,
        out_type=jax.ShapeDtypeStruct((B, D), jnp.float32),
        mesh=mesh,
        scratch_types=[
            pltpu.VMEM((K, IDX_W), jnp.int32),
            pltpu.VMEM((CHUNK, D), jnp.float32),
            pltpu.SemaphoreType.DMA,
        ],
        compiler_params=pltpu.CompilerParams(use_tc_tiling_on_sc=False),
    )
    def body(x_hbm, table_hbm, out_hbm, idx_v, rows_v, gsem):
        wid = lax.axis_index("s") * NC + lax.axis_index("c")
        chunk0 = wid * n_chunks

        def chunk_body(i, carry):
            gid = chunk0 + i
            pltpu.sync_copy(x_hbm.at[pl.ds(gid * K, K), :], idx_v)
            descs = []
            for j in range(K):
                descs.append(
                    pltpu.async_copy(
                        table_hbm.at[idx_v.at[j]],
                        rows_v.at[pl.ds(j * IDX_W, IDX_W), :],
                        gsem,
                    )
                )
            for d in descs:
                d.wait()
            pltpu.sync_copy(rows_v, out_hbm.at[pl.ds(gid * CHUNK, CHUNK), :])
            return carry

        lax.fori_loop(0, n_chunks, chunk_body, 0)

    return body


def kernel(x, table):
    B0, B1 = x.shape
    V, D = table.shape
    B = B0 * B1
    x_flat = x.reshape(B // IDX_W, IDX_W).astype(jnp.int32)
    out = _lookup_kernel(B, V, D)(x_flat, table)
    return out.reshape(B0, B1, D)


# speedup vs baseline: 5.0263x; 1.0155x over previous
import functools

import jax
import jax.numpy as jnp
from jax import lax
from jax.experimental import pallas as pl
from jax.experimental.pallas import tpu as pltpu
from jax.experimental.pallas import tpu_sc as plsc

NC = 2
NS = 16
NW = NC * NS

IDX_W = 128
K = 8
CHUNK = K * IDX_W
IB = 10


def _lookup_kernel(B, V, D):
    b_per_w = B // NW
    n_chunks = b_per_w // CHUNK
    n_blocks = n_chunks // IB
    mesh = plsc.VectorSubcoreMesh(core_axis_name="c", subcore_axis_name="s")

    @functools.partial(
        pl.kernel,
        out_type=jax.ShapeDtypeStruct((B, D), jnp.float32),
        mesh=mesh,
        scratch_types=[
            pltpu.VMEM((2, IB * K, IDX_W), jnp.int32),
            pltpu.VMEM((2, CHUNK, D), jnp.float32),
            pltpu.SemaphoreType.DMA,
            pltpu.SemaphoreType.DMA,
            pltpu.SemaphoreType.DMA,
            pltpu.SemaphoreType.DMA,
            pltpu.SemaphoreType.DMA,
            pltpu.SemaphoreType.DMA,
        ],
        compiler_params=pltpu.CompilerParams(use_tc_tiling_on_sc=False),
    )
    def body(x_hbm, table_hbm, out_hbm, ibuf, rows_v, is0, is1, os0, os1,
             gs0, gs1):
        isem = (is0, is1)
        osem = (os0, os1)
        gsem = (gs0, gs1)
        wid = lax.axis_index("s") * NC + lax.axis_index("c")
        chunk0 = wid * n_chunks

        def idx_rows(block):
            return pl.ds((chunk0 + block * IB) * K, IB * K)

        def wait_idx(p):
            pltpu.make_async_copy(
                x_hbm.at[pl.ds(0, IB * K), :], ibuf.at[p], isem[p]).wait()

        def wait_out(p):
            pltpu.make_async_copy(
                out_hbm.at[pl.ds(0, CHUNK), :], rows_v.at[p], osem[p]).wait()

        for p in range(2):
            pltpu.async_copy(x_hbm.at[idx_rows(p)], ibuf.at[p], isem[p])
            pltpu.async_copy(
                rows_v.at[p],
                out_hbm.at[pl.ds((chunk0 + p) * CHUNK, CHUNK), :],
                osem[p],
            )

        def outer(t, carry):
            for mb in range(2):
                block = 2 * t + mb
                wait_idx(mb)

                def inner(u, c2):
                    for cb in range(2):
                        wc = block * IB + 2 * u + cb
                        row0 = (2 * u + cb) * K
                        gid = chunk0 + wc
                        wait_out(cb)
                        descs = []
                        for j in range(K):
                            descs.append(pltpu.async_copy(
                                table_hbm.at[ibuf.at[mb].at[row0 + j]],
                                rows_v.at[cb].at[pl.ds(j * IDX_W, IDX_W), :],
                                gsem[cb],
                            ))
                        for d_ in descs:
                            d_.wait()
                        pltpu.async_copy(
                            rows_v.at[cb],
                            out_hbm.at[pl.ds(gid * CHUNK, CHUNK), :],
                            osem[cb],
                        )
                    return c2

                lax.fori_loop(0, IB // 2, inner, 0)
                nxt = block + 2
                nxt = lax.select(nxt < n_blocks, nxt, n_blocks - 1)
                pltpu.async_copy(x_hbm.at[idx_rows(nxt)], ibuf.at[mb],
                                 isem[mb])
            return carry

        lax.fori_loop(0, n_blocks // 2, outer, 0)

        for p in range(2):
            wait_idx(p)
            wait_out(p)

    return body


def kernel(x, table):
    B0, B1 = x.shape
    V, D = table.shape
    B = B0 * B1
    x_flat = x.reshape(B // IDX_W, IDX_W).astype(jnp.int32)
    out = _lookup_kernel(B, V, D)(x_flat, table)
    return out.reshape(B0, B1, D)
